# baseline (speedup 1.0000x reference)
import functools
import jax
import jax.numpy as jnp
from jax import lax
from jax.experimental import pallas as pl
from jax.experimental.pallas import tpu as pltpu
from jax.experimental.pallas import tpu_sc as plsc

_NT, _NY, _NX = 5, 32, 32
_NB = _NY * _NX
_D9 = [(dy, dx) for dy in (-1, 0, 1) for dx in (-1, 0, 1)]
_F25 = [(fy, fx) for fy in (-2, -1, 0, 1, 2) for fx in (-2, -1, 0, 1, 2)]
_NQ = _NT * _NB


def _softplus10(z):
    return jax.nn.softplus(10.0 * z) / 10.0


def _encoder_body(x_ref, w1_ref, w2_ref, w3_ref, out_ref):
    lane = jax.lax.broadcasted_iota(jnp.int32, (1, _NB), 1)
    yy = lane // _NX
    xx = lane % _NX

    def conv(h, w_ref):
        acc = None
        for t, (dy, dx) in enumerate(_D9):
            s = dy * _NX + dx
            rolled = h if s == 0 else jnp.roll(h, -s, axis=1)
            m = ((yy + dy >= 0) & (yy + dy < _NY)
                 & (xx + dx >= 0) & (xx + dx < _NX))
            hs = jnp.where(m, rolled, 0.0)
            p = jnp.dot(w_ref[t], hs, preferred_element_type=jnp.float32)
            acc = p if acc is None else acc + p
        return acc

    h = conv(jax.nn.relu(x_ref[...]), w1_ref)
    h = conv(jax.nn.relu(h), w2_ref)
    out_ref[...] = conv(h, w3_ref)


def _coeff_body(ks_ref, m1_ref, m2_ref, ga_ref, vx_ref, vy_ref,
                wd_ref, wo_ref):
    lane9 = jax.lax.broadcasted_iota(jnp.int32, (9, _NB), 1)
    xn = lane9 % _NX
    rowc = jax.lax.broadcasted_iota(jnp.int32, (_NX, _NB), 0)
    ln = jax.lax.broadcasted_iota(jnp.int32, (_NX, _NB), 1) % _NX
    pmask = {fx: rowc == ((ln + fx) % _NX) for fx in range(-2, 3)}
    for k in range(_NT):
        kap = _softplus10(ks_ref[k:k + 1, :])
        gam = _softplus10(ga_ref[k:k + 1, :])
        vxk = vx_ref[k:k + 1, :]
        vyk = vy_ref[k:k + 1, :]
        m1 = m1_ref[k:k + 1, :]
        m2 = m2_ref[k:k + 1, :]
        a = gam + vxk * vxk
        bb = vxk * vyk
        cc = gam + vyk * vyk
        kap2 = kap * kap
        cmap = {
            (0, 0): 1.0 + kap2 + 2.0 * a + 2.0 * cc,
            (0, 1): -a + 0.5 * m1,
            (0, -1): -a - 0.5 * m1,
            (1, 0): -cc + 0.5 * m2,
            (-1, 0): -cc - 0.5 * m2,
            (1, 1): -0.5 * bb,
            (-1, -1): -0.5 * bb,
            (1, -1): 0.5 * bb,
            (-1, 1): 0.5 * bb,
        }
        cstack = jnp.concatenate([cmap[d] for d in _D9], axis=0)
        wo_chunks = []
        for fy in (-1, 0, 1):
            acc = None
            for fx in (-1, 0, 1):
                term = jnp.where(pmask[fx], -cmap[(fy, fx)], 0.0)
                acc = term if acc is None else acc + term
            wo_chunks.append(acc)
        wo_ref[k] = jnp.transpose(jnp.concatenate(wo_chunks, axis=0))
        g = {f: None for f in _F25}
        for di, (dy, dx) in enumerate(_D9):
            s = dy * _NX + dx
            r0 = cstack if s == 0 else jnp.roll(cstack, -s, axis=1)
            if dx == 0:
                sh = r0
            else:
                r1 = jnp.roll(cstack, -(s - dx * _NX), axis=1)
                wrap = (xn + dx >= _NX) if dx > 0 else (xn + dx < 0)
                sh = jnp.where(wrap, r1, r0)
            for ei, (ey, ex) in enumerate(_D9):
                f = (dy + ey, dx + ex)
                term = cstack[di:di + 1, :] * sh[ei:ei + 1, :]
                g[f] = term if g[f] is None else g[f] + term
        if 0 < k < _NT - 1:
            g[(0, 0)] = g[(0, 0)] + 1.0
        wd_chunks = []
        for fy in (-2, -1, 0, 1, 2):
            acc = None
            for fx in (-2, -1, 0, 1, 2):
                term = jnp.where(pmask[fx], g[(fy, fx)], 0.0)
                acc = term if acc is None else acc + term
            wd_chunks.append(acc)
        wd_ref[k] = jnp.transpose(jnp.concatenate(wd_chunks, axis=0))


def _strip_segments(fys):
    segs = []
    for ry in range(_NY):
        cyts = [(ry + fy) % _NY for fy in fys]
        runs = []
        k0 = 0
        for k in range(1, len(cyts) + 1):
            if k == len(cyts) or cyts[k] != cyts[k - 1] + 1:
                runs.append((k0 * _NX, cyts[k0] * _NX, (k - k0) * _NX))
                k0 = k
        segs.append(runs)
    return segs


_SEG5 = _strip_segments(range(-2, 3))
_SEG3 = _strip_segments(range(-1, 2))

_NWORK = 32
_CHUNK = 128


def _sc_tables():
    src, dst = [], []
    for i in range(_NT):
        for r in range(_NB):
            qr = i * _NB + r
            ry = r // _NX
            for (w0, b0, wid) in _SEG5[ry]:
                for c in range(wid // _NX):
                    src.append((i * _NB + r) * 5 + w0 // _NX + c)
                    dst.append(qr * (_NQ // _NX) + (i * _NB + b0) // _NX + c)
            for j in (i - 1, i + 1):
                if 0 <= j < _NT:
                    for (w0, b0, wid) in _SEG3[ry]:
                        for c in range(wid // _NX):
                            src.append(_NT * _NB * 5 + (j * _NB + r) * 3
                                       + w0 // _NX + c)
                            dst.append(qr * (_NQ // _NX)
                                       + (j * _NB + b0) // _NX + c)
    n = len(src)
    per_w = -(-n // (_NWORK * _CHUNK))
    total = _NWORK * per_w * _CHUNK
    src += [src[-1]] * (total - n)
    dst += [dst[-1]] * (total - n)
    import numpy as np
    return (np.asarray(src, np.int32).reshape(_NWORK, per_w, _CHUNK),
            np.asarray(dst, np.int32).reshape(_NWORK, per_w, _CHUNK),
            per_w)


_SRC_TAB, _DST_TAB, _PER_W = _sc_tables()


def _compl_runs():
    out = []
    for i in range(_NT):
        rows = []
        for ry in range(_NY):
            band = []
            for (w0, b0, wid) in _SEG5[ry]:
                band.append((i * _NB + b0, wid))
            for j in (i - 1, i + 1):
                if 0 <= j < _NT:
                    for (w0, b0, wid) in _SEG3[ry]:
                        band.append((j * _NB + b0, wid))
            band.sort()
            comp = []
            pos = 0
            for (b, w) in band:
                if b > pos:
                    comp.append((pos, b - pos))
                pos = b + w
            if pos < _NQ:
                comp.append((pos, _NQ - pos))
            rows.append(comp)
        out.append(rows)
    return out


_COMPL = _compl_runs()


def _sc_scatter_body(w32_ref, sidx_ref, didx_ref, q_ref,
                     idx_s, idx_d, rows, sem):
    wid = lax.axis_index("s") * 2 + lax.axis_index("c")
    pltpu.sync_copy(sidx_ref.at[wid], idx_s)
    pltpu.sync_copy(didx_ref.at[wid], idx_d)
    descs = []
    for j in range(_PER_W):
        descs.append(pltpu.async_copy(
            w32_ref.at[idx_s.at[j]],
            rows.at[pl.ds(j * _CHUNK, _CHUNK)], sem))
    for d in descs:
        d.wait()
    descs = []
    for j in range(_PER_W):
        descs.append(pltpu.async_copy(
            rows.at[pl.ds(j * _CHUNK, _CHUNK)],
            q_ref.at[idx_d.at[j]], sem))
    for d in descs:
        d.wait()


def _sc_scatter(w32, sidx, didx, q_ref, interpret=False):
    mesh = plsc.VectorSubcoreMesh(core_axis_name="c", subcore_axis_name="s")
    kern = pl.kernel(
        _sc_scatter_body,
        out_type=(),
        mesh=mesh,
        compiler_params=pltpu.CompilerParams(use_tc_tiling_on_sc=False),
        scratch_types=[
            pltpu.VMEM((_PER_W, _CHUNK), jnp.int32),
            pltpu.VMEM((_PER_W, _CHUNK), jnp.int32),
            pltpu.VMEM((_PER_W * _CHUNK, _NX), jnp.float32),
            pltpu.SemaphoreType.DMA,
        ],
        interpret=interpret,
    )
    kern(w32, sidx, didx, q_ref)


def _memset_body(out_ref):
    out_ref[...] = jnp.zeros_like(out_ref)


def _memset_q(interpret=False):
    return pl.pallas_call(
        _memset_body,
        grid=(_NT,),
        out_specs=pl.BlockSpec((1, _NB, _NQ), lambda i: (0, i, 0)),
        out_shape=jax.ShapeDtypeStruct((1, _NQ, _NQ), jnp.float32),
        interpret=interpret,
    )()


def _build_q(xf, w1r, w2r, w3r, interpret=False):
    params = pl.pallas_call(
        _encoder_body,
        out_shape=jax.ShapeDtypeStruct((6 * _NT, _NB), jnp.float32),
        interpret=interpret,
    )(xf, w1r, w2r, w3r)

    def scramble(p5):
        return p5.reshape(_NT * _NB).reshape(_NB, _NT).T

    ks = scramble(params[0:_NT])
    m1s = scramble(params[_NT:2 * _NT])
    m2s = scramble(params[2 * _NT:3 * _NT])
    ga = params[3 * _NT:4 * _NT]
    vx = params[4 * _NT:5 * _NT]
    vy = params[5 * _NT:6 * _NT]

    wd, wo = pl.pallas_call(
        _coeff_body,
        out_shape=(
            jax.ShapeDtypeStruct((_NT, _NB, 5 * _NX), jnp.float32),
            jax.ShapeDtypeStruct((_NT, _NB, 3 * _NX), jnp.float32),
        ),
        interpret=interpret,
    )(ks, m1s, m2s, ga, vx, vy)

    w32 = jnp.concatenate(
        [wd.reshape(-1, _NX), wo.reshape(-1, _NX)], axis=0)
    q0 = _memset_q(interpret=interpret).reshape(_NQ * (_NQ // _NX), _NX)
    q_ref = jax.new_ref(q0)
    _sc_scatter(w32, jnp.asarray(_SRC_TAB), jnp.asarray(_DST_TAB), q_ref,
                interpret=interpret)
    return jax.freeze(q_ref).reshape(_NQ, _NQ)


def kernel(x, kappa, m, H, W1, W2, W3):
    del kappa, m, H
    w1r = jnp.transpose(W1.reshape(W1.shape[0], W1.shape[1], 9), (2, 0, 1))
    w2r = jnp.transpose(W2.reshape(W2.shape[0], W2.shape[1], 9), (2, 0, 1))
    w3r = jnp.transpose(W3.reshape(W3.shape[0], W3.shape[1], 9), (2, 0, 1))
    qs = []
    for b in range(x.shape[0]):
        xf = x[b].reshape(_NT, _NB)
        qs.append(_build_q(xf, w1r, w2r, w3r))
    return jnp.stack(qs)


# speedup vs baseline: 1.2709x; 1.2709x over previous
import functools
import jax
import jax.numpy as jnp
from jax import lax
from jax.experimental import pallas as pl
from jax.experimental.pallas import tpu as pltpu
from jax.experimental.pallas import tpu_sc as plsc

_NT, _NY, _NX = 5, 32, 32
_NB = _NY * _NX
_D9 = [(dy, dx) for dy in (-1, 0, 1) for dx in (-1, 0, 1)]
_F25 = [(fy, fx) for fy in (-2, -1, 0, 1, 2) for fx in (-2, -1, 0, 1, 2)]
_NQ = _NT * _NB


def _softplus10(z):
    return jax.nn.softplus(10.0 * z) / 10.0


def _encoder_body(x_ref, w1_ref, w2_ref, w3_ref, out_ref):
    lane = jax.lax.broadcasted_iota(jnp.int32, (1, _NB), 1)
    yy = lane // _NX
    xx = lane % _NX

    def conv(h, w_ref):
        acc = None
        for t, (dy, dx) in enumerate(_D9):
            s = dy * _NX + dx
            rolled = h if s == 0 else jnp.roll(h, -s, axis=1)
            m = ((yy + dy >= 0) & (yy + dy < _NY)
                 & (xx + dx >= 0) & (xx + dx < _NX))
            hs = jnp.where(m, rolled, 0.0)
            p = jnp.dot(w_ref[t], hs, preferred_element_type=jnp.float32)
            acc = p if acc is None else acc + p
        return acc

    h = conv(jax.nn.relu(x_ref[...]), w1_ref)
    h = conv(jax.nn.relu(h), w2_ref)
    out_ref[...] = conv(h, w3_ref)


def _coeff_body(ks_ref, m1_ref, m2_ref, ga_ref, vx_ref, vy_ref,
                wd_ref, wo_ref):
    lane9 = jax.lax.broadcasted_iota(jnp.int32, (9, _NB), 1)
    xn = lane9 % _NX
    rowc = jax.lax.broadcasted_iota(jnp.int32, (_NX, _NB), 0)
    ln = jax.lax.broadcasted_iota(jnp.int32, (_NX, _NB), 1) % _NX
    pmask = {fx: rowc == ((ln + fx) % _NX) for fx in range(-2, 3)}
    for k in range(_NT):
        kap = _softplus10(ks_ref[k:k + 1, :])
        gam = _softplus10(ga_ref[k:k + 1, :])
        vxk = vx_ref[k:k + 1, :]
        vyk = vy_ref[k:k + 1, :]
        m1 = m1_ref[k:k + 1, :]
        m2 = m2_ref[k:k + 1, :]
        a = gam + vxk * vxk
        bb = vxk * vyk
        cc = gam + vyk * vyk
        kap2 = kap * kap
        cmap = {
            (0, 0): 1.0 + kap2 + 2.0 * a + 2.0 * cc,
            (0, 1): -a + 0.5 * m1,
            (0, -1): -a - 0.5 * m1,
            (1, 0): -cc + 0.5 * m2,
            (-1, 0): -cc - 0.5 * m2,
            (1, 1): -0.5 * bb,
            (-1, -1): -0.5 * bb,
            (1, -1): 0.5 * bb,
            (-1, 1): 0.5 * bb,
        }
        cstack = jnp.concatenate([cmap[d] for d in _D9], axis=0)
        wo_chunks = []
        for fy in (-1, 0, 1):
            acc = None
            for fx in (-1, 0, 1):
                term = jnp.where(pmask[fx], -cmap[(fy, fx)], 0.0)
                acc = term if acc is None else acc + term
            wo_chunks.append(acc)
        wo_ref[k] = jnp.transpose(jnp.concatenate(wo_chunks, axis=0))
        g = {f: None for f in _F25}
        for di, (dy, dx) in enumerate(_D9):
            s = dy * _NX + dx
            r0 = cstack if s == 0 else jnp.roll(cstack, -s, axis=1)
            if dx == 0:
                sh = r0
            else:
                r1 = jnp.roll(cstack, -(s - dx * _NX), axis=1)
                wrap = (xn + dx >= _NX) if dx > 0 else (xn + dx < 0)
                sh = jnp.where(wrap, r1, r0)
            for ei, (ey, ex) in enumerate(_D9):
                f = (dy + ey, dx + ex)
                term = cstack[di:di + 1, :] * sh[ei:ei + 1, :]
                g[f] = term if g[f] is None else g[f] + term
        if 0 < k < _NT - 1:
            g[(0, 0)] = g[(0, 0)] + 1.0
        wd_chunks = []
        for fy in (-2, -1, 0, 1, 2):
            acc = None
            for fx in (-2, -1, 0, 1, 2):
                term = jnp.where(pmask[fx], g[(fy, fx)], 0.0)
                acc = term if acc is None else acc + term
            wd_chunks.append(acc)
        wd_ref[k] = jnp.transpose(jnp.concatenate(wd_chunks, axis=0))


def _strip_segments(fys):
    segs = []
    for ry in range(_NY):
        cyts = [(ry + fy) % _NY for fy in fys]
        runs = []
        k0 = 0
        for k in range(1, len(cyts) + 1):
            if k == len(cyts) or cyts[k] != cyts[k - 1] + 1:
                runs.append((k0 * _NX, cyts[k0] * _NX, (k - k0) * _NX))
                k0 = k
        segs.append(runs)
    return segs


_SEG5 = _strip_segments(range(-2, 3))
_SEG3 = _strip_segments(range(-1, 2))

_NWORK = 32
_CHUNK = 128


def _sc_tables():
    rows_per_w = _NQ // _NWORK
    per_lists = [([], []) for _ in range(_NWORK)]
    for i in range(_NT):
        for r in range(_NB):
            qr = i * _NB + r
            src, dst = per_lists[qr // rows_per_w]
            ry = r // _NX
            for (w0, b0, wid) in _SEG5[ry]:
                for c in range(wid // _NX):
                    src.append((i * _NB + r) * 5 + w0 // _NX + c)
                    dst.append(qr * (_NQ // _NX) + (i * _NB + b0) // _NX + c)
            for j in (i - 1, i + 1):
                if 0 <= j < _NT:
                    for (w0, b0, wid) in _SEG3[ry]:
                        for c in range(wid // _NX):
                            src.append(_NT * _NB * 5 + (j * _NB + r) * 3
                                       + w0 // _NX + c)
                            dst.append(qr * (_NQ // _NX)
                                       + (j * _NB + b0) // _NX + c)
    per_w = -(-max(len(s) for s, _ in per_lists) // _CHUNK)
    import numpy as np
    srcs, dsts = [], []
    for (src, dst) in per_lists:
        pad = per_w * _CHUNK - len(src)
        srcs.append(np.asarray(src + [src[-1]] * pad, np.int32))
        dsts.append(np.asarray(dst + [dst[-1]] * pad, np.int32))
    return (np.stack(srcs).reshape(_NWORK, per_w, _CHUNK),
            np.stack(dsts).reshape(_NWORK, per_w, _CHUNK),
            per_w)


_SRC_TAB, _DST_TAB, _PER_W = _sc_tables()


def _compl_runs():
    out = []
    for i in range(_NT):
        rows = []
        for ry in range(_NY):
            band = []
            for (w0, b0, wid) in _SEG5[ry]:
                band.append((i * _NB + b0, wid))
            for j in (i - 1, i + 1):
                if 0 <= j < _NT:
                    for (w0, b0, wid) in _SEG3[ry]:
                        band.append((j * _NB + b0, wid))
            band.sort()
            comp = []
            pos = 0
            for (b, w) in band:
                if b > pos:
                    comp.append((pos, b - pos))
                pos = b + w
            if pos < _NQ:
                comp.append((pos, _NQ - pos))
            rows.append(comp)
        out.append(rows)
    return out


_COMPL = _compl_runs()


_ZROWS = 1024
_Z_PER_W = (_NQ // _NWORK) * (_NQ // _NX) // _ZROWS


def _sc_assemble_body(w32_ref, sidx_ref, didx_ref, z_ref, out_ref,
                      idx_s, idx_d, rows, zbuf, sem):
    wid = lax.axis_index("s") * 2 + lax.axis_index("c")
    pltpu.sync_copy(sidx_ref.at[wid], idx_s)
    pltpu.sync_copy(didx_ref.at[wid], idx_d)
    pltpu.sync_copy(z_ref, zbuf)
    base = wid * (_Z_PER_W * _ZROWS)
    descs = []
    for t in range(_Z_PER_W):
        descs.append(pltpu.async_copy(
            zbuf, out_ref.at[pl.ds(base + t * _ZROWS, _ZROWS)], sem))
    for j in range(_PER_W):
        descs.append(pltpu.async_copy(
            w32_ref.at[idx_s.at[j]],
            rows.at[pl.ds(j * _CHUNK, _CHUNK)], sem))
    for d in descs:
        d.wait()
    descs = []
    for j in range(_PER_W):
        descs.append(pltpu.async_copy(
            rows.at[pl.ds(j * _CHUNK, _CHUNK)],
            out_ref.at[idx_d.at[j]], sem))
    for d in descs:
        d.wait()


def _sc_assemble(w32, sidx, didx, zeros, interpret=False):
    mesh = plsc.VectorSubcoreMesh(core_axis_name="c", subcore_axis_name="s")
    kern = pl.kernel(
        _sc_assemble_body,
        out_type=jax.ShapeDtypeStruct((_NQ * (_NQ // _NX), _NX), jnp.float32),
        mesh=mesh,
        compiler_params=pltpu.CompilerParams(use_tc_tiling_on_sc=False),
        scratch_types=[
            pltpu.VMEM((_PER_W, _CHUNK), jnp.int32),
            pltpu.VMEM((_PER_W, _CHUNK), jnp.int32),
            pltpu.VMEM((_PER_W * _CHUNK, _NX), jnp.float32),
            pltpu.VMEM((_ZROWS, _NX), jnp.float32),
            pltpu.SemaphoreType.DMA,
        ],
        interpret=interpret,
    )
    return kern(w32, sidx, didx, zeros)


def _build_q(xf, w1r, w2r, w3r, interpret=False):
    params = pl.pallas_call(
        _encoder_body,
        out_shape=jax.ShapeDtypeStruct((6 * _NT, _NB), jnp.float32),
        interpret=interpret,
    )(xf, w1r, w2r, w3r)

    def scramble(p5):
        return p5.reshape(_NT * _NB).reshape(_NB, _NT).T

    ks = scramble(params[0:_NT])
    m1s = scramble(params[_NT:2 * _NT])
    m2s = scramble(params[2 * _NT:3 * _NT])
    ga = params[3 * _NT:4 * _NT]
    vx = params[4 * _NT:5 * _NT]
    vy = params[5 * _NT:6 * _NT]

    wd, wo = pl.pallas_call(
        _coeff_body,
        out_shape=(
            jax.ShapeDtypeStruct((_NT, _NB, 5 * _NX), jnp.float32),
            jax.ShapeDtypeStruct((_NT, _NB, 3 * _NX), jnp.float32),
        ),
        interpret=interpret,
    )(ks, m1s, m2s, ga, vx, vy)

    w32 = jnp.concatenate(
        [wd.reshape(-1, _NX), wo.reshape(-1, _NX)], axis=0)
    zeros = jnp.zeros((_ZROWS, _NX), jnp.float32)
    q32 = _sc_assemble(w32, jnp.asarray(_SRC_TAB), jnp.asarray(_DST_TAB),
                       zeros, interpret=interpret)
    return q32.reshape(_NQ, _NQ)


def kernel(x, kappa, m, H, W1, W2, W3):
    del kappa, m, H
    w1r = jnp.transpose(W1.reshape(W1.shape[0], W1.shape[1], 9), (2, 0, 1))
    w2r = jnp.transpose(W2.reshape(W2.shape[0], W2.shape[1], 9), (2, 0, 1))
    w3r = jnp.transpose(W3.reshape(W3.shape[0], W3.shape[1], 9), (2, 0, 1))
    qs = []
    for b in range(x.shape[0]):
        xf = x[b].reshape(_NT, _NB)
        qs.append(_build_q(xf, w1r, w2r, w3r))
    return jnp.stack(qs)


# speedup vs baseline: 5.4505x; 4.2888x over previous
import jax
import jax.numpy as jnp
from jax.experimental import pallas as pl

_NT, _NY, _NX = 5, 32, 32
_NB = _NY * _NX
_D9 = [(dy, dx) for dy in (-1, 0, 1) for dx in (-1, 0, 1)]
_F25 = [(fy, fx) for fy in (-2, -1, 0, 1, 2) for fx in (-2, -1, 0, 1, 2)]


def _softplus10(z):
    return jax.nn.softplus(10.0 * z) / 10.0


def _encoder_body(x_ref, w1_ref, w2_ref, w3_ref, out_ref):
    lane = jax.lax.broadcasted_iota(jnp.int32, (1, _NB), 1)
    yy = lane // _NX
    xx = lane % _NX

    def conv(h, w_ref):
        acc = None
        for t, (dy, dx) in enumerate(_D9):
            s = dy * _NX + dx
            rolled = h if s == 0 else jnp.roll(h, -s, axis=1)
            m = ((yy + dy >= 0) & (yy + dy < _NY)
                 & (xx + dx >= 0) & (xx + dx < _NX))
            hs = jnp.where(m, rolled, 0.0)
            p = jnp.dot(w_ref[t], hs, preferred_element_type=jnp.float32)
            acc = p if acc is None else acc + p
        return acc

    h = conv(jax.nn.relu(x_ref[...]), w1_ref)
    h = conv(jax.nn.relu(h), w2_ref)
    out_ref[...] = conv(h, w3_ref)


def _coeff_body(ks_ref, m1_ref, m2_ref, ga_ref, vx_ref, vy_ref,
                wd_ref, wo_ref):
    lane9 = jax.lax.broadcasted_iota(jnp.int32, (9, _NB), 1)
    xn = lane9 % _NX
    rowc = jax.lax.broadcasted_iota(jnp.int32, (_NX, _NB), 0)
    ln = jax.lax.broadcasted_iota(jnp.int32, (_NX, _NB), 1) % _NX
    pmask = {fx: rowc == ((ln + fx) % _NX) for fx in range(-2, 3)}
    for k in range(_NT):
        kap = _softplus10(ks_ref[k:k + 1, :])
        gam = _softplus10(ga_ref[k:k + 1, :])
        vxk = vx_ref[k:k + 1, :]
        vyk = vy_ref[k:k + 1, :]
        m1 = m1_ref[k:k + 1, :]
        m2 = m2_ref[k:k + 1, :]
        a = gam + vxk * vxk
        bb = vxk * vyk
        cc = gam + vyk * vyk
        kap2 = kap * kap
        cmap = {
            (0, 0): 1.0 + kap2 + 2.0 * a + 2.0 * cc,
            (0, 1): -a + 0.5 * m1,
            (0, -1): -a - 0.5 * m1,
            (1, 0): -cc + 0.5 * m2,
            (-1, 0): -cc - 0.5 * m2,
            (1, 1): -0.5 * bb,
            (-1, -1): -0.5 * bb,
            (1, -1): 0.5 * bb,
            (-1, 1): 0.5 * bb,
        }
        cstack = jnp.concatenate([cmap[d] for d in _D9], axis=0)
        wo_chunks = []
        for fy in (-1, 0, 1):
            acc = None
            for fx in (-1, 0, 1):
                term = jnp.where(pmask[fx], -cmap[(fy, fx)], 0.0)
                acc = term if acc is None else acc + term
            wo_chunks.append(acc)
        wo_ref[k] = jnp.transpose(jnp.concatenate(wo_chunks, axis=0))
        g = {f: None for f in _F25}
        for di, (dy, dx) in enumerate(_D9):
            s = dy * _NX + dx
            r0 = cstack if s == 0 else jnp.roll(cstack, -s, axis=1)
            if dx == 0:
                sh = r0
            else:
                r1 = jnp.roll(cstack, -(s - dx * _NX), axis=1)
                wrap = (xn + dx >= _NX) if dx > 0 else (xn + dx < 0)
                sh = jnp.where(wrap, r1, r0)
            for ei, (ey, ex) in enumerate(_D9):
                f = (dy + ey, dx + ex)
                term = cstack[di:di + 1, :] * sh[ei:ei + 1, :]
                g[f] = term if g[f] is None else g[f] + term
        if 0 < k < _NT - 1:
            g[(0, 0)] = g[(0, 0)] + 1.0
        wd_chunks = []
        for fy in (-2, -1, 0, 1, 2):
            acc = None
            for fx in (-2, -1, 0, 1, 2):
                term = jnp.where(pmask[fx], g[(fy, fx)], 0.0)
                acc = term if acc is None else acc + term
            wd_chunks.append(acc)
        wd_ref[k] = jnp.transpose(jnp.concatenate(wd_chunks, axis=0))


def _strip_segments(fys):
    segs = []
    for ry in range(_NY):
        cyts = [(ry + fy) % _NY for fy in fys]
        runs = []
        k0 = 0
        for k in range(1, len(cyts) + 1):
            if k == len(cyts) or cyts[k] != cyts[k - 1] + 1:
                runs.append((k0 * _NX, cyts[k0] * _NX, (k - k0) * _NX))
                k0 = k
        segs.append(runs)
    return segs


_SEG5 = _strip_segments(range(-2, 3))
_SEG3 = _strip_segments(range(-1, 2))


def _assemble_body(wd_ref, wo_ref, out_ref):
    i = pl.program_id(0)

    out_ref[...] = jnp.zeros_like(out_ref)

    for j in range(_NT):
        base = j * _NB

        @pl.when(i == j)
        def _(j=j, base=base):
            for ry in range(_NY):
                rr = pl.ds(ry * _NX, _NX)
                for (w0, b0, wid) in _SEG5[ry]:
                    out_ref[0, rr, pl.ds(base + b0, wid)] = (
                        wd_ref[0, rr, pl.ds(w0, wid)])

        @pl.when((i == j - 1) | (i == j + 1))
        def _(j=j, base=base):
            for ry in range(_NY):
                rr = pl.ds(ry * _NX, _NX)
                for (w0, b0, wid) in _SEG3[ry]:
                    out_ref[0, rr, pl.ds(base + b0, wid)] = (
                        wo_ref[j, rr, pl.ds(w0, wid)])


def _build_q(xf, w1r, w2r, w3r):
    params = pl.pallas_call(
        _encoder_body,
        out_shape=jax.ShapeDtypeStruct((6 * _NT, _NB), jnp.float32),
    )(xf, w1r, w2r, w3r)

    def scramble(p5):
        return p5.reshape(_NT * _NB).reshape(_NB, _NT).T

    ks = scramble(params[0:_NT])
    m1s = scramble(params[_NT:2 * _NT])
    m2s = scramble(params[2 * _NT:3 * _NT])
    ga = params[3 * _NT:4 * _NT]
    vx = params[4 * _NT:5 * _NT]
    vy = params[5 * _NT:6 * _NT]

    wd, wo = pl.pallas_call(
        _coeff_body,
        out_shape=(
            jax.ShapeDtypeStruct((_NT, _NB, 5 * _NX), jnp.float32),
            jax.ShapeDtypeStruct((_NT, _NB, 3 * _NX), jnp.float32),
        ),
    )(ks, m1s, m2s, ga, vx, vy)

    q = pl.pallas_call(
        _assemble_body,
        grid=(_NT,),
        in_specs=[
            pl.BlockSpec((1, _NB, 5 * _NX), lambda i: (i, 0, 0)),
            pl.BlockSpec((_NT, _NB, 3 * _NX), lambda i: (0, 0, 0)),
        ],
        out_specs=pl.BlockSpec((1, _NB, _NT * _NB), lambda i: (0, i, 0)),
        out_shape=jax.ShapeDtypeStruct((1, _NT * _NB, _NT * _NB), jnp.float32),
    )(wd, wo)
    return q[0]


def kernel(x, kappa, m, H, W1, W2, W3):
    del kappa, m, H
    w1r = jnp.transpose(W1.reshape(W1.shape[0], W1.shape[1], 9), (2, 0, 1))
    w2r = jnp.transpose(W2.reshape(W2.shape[0], W2.shape[1], 9), (2, 0, 1))
    w3r = jnp.transpose(W3.reshape(W3.shape[0], W3.shape[1], 9), (2, 0, 1))
    qs = []
    for b in range(x.shape[0]):
        xf = x[b].reshape(_NT, _NB)
        qs.append(_build_q(xf, w1r, w2r, w3r))
    return jnp.stack(qs)
